# baseline (speedup 1.0000x reference)
import jax
import jax.numpy as jnp
import numpy as np
from jax import lax
from jax.experimental import pallas as pl
from jax.experimental.pallas import tpu as pltpu
from jax.experimental.pallas import tpu_sc as plsc

N = 10000
E = 320000
D = 128
H = 8
C = D // H
NC = 2
NS = 16
NW = NC * NS
EC = 128
EPAD = 327680
CH = EPAD // EC
SUPK = 8
S1 = CH // (NW * SUPK)
K2N = CH // NW
NPAD = 10240
RPS = NPAD // NS
DR = 128
DENW = NPAD * H

BLK = 1000
BLKE = 8192

_KRON = np.kron(np.eye(H, dtype=np.float32), np.ones((C, 1), np.float32))
_PSEL = np.zeros((16, 128), np.float32)
for _f in range(8):
    for _l in range(16):
        _PSEL[2 * _f + (_l >= 8), _f * 16 + _l] = 1.0


def _pre_body(x_ref, xn_ref, w_ref, vs_ref, vd_ref, hs_ref, asat_ref):
    x = x_ref[...]
    hs_ref[...] = jnp.dot(x, w_ref[...], preferred_element_type=jnp.float32)
    asat_ref[...] = (
        jnp.dot(x, vs_ref[...], preferred_element_type=jnp.float32)
        + jnp.dot(xn_ref[...], vd_ref[...], preferred_element_type=jnp.float32))


def _pre(x, xn, w, vs, vd):
    return pl.pallas_call(
        _pre_body,
        grid=(N // 2000,),
        in_specs=[
            pl.BlockSpec((2000, D), lambda i: (i, 0)),
            pl.BlockSpec((2000, D), lambda i: (i, 0)),
            pl.BlockSpec((D, D), lambda i: (0, 0)),
            pl.BlockSpec((D, 2 * H), lambda i: (0, 0)),
            pl.BlockSpec((D, 2 * H), lambda i: (0, 0)),
        ],
        out_specs=[
            pl.BlockSpec((2000, D), lambda i: (i, 0)),
            pl.BlockSpec((2000, 2 * H), lambda i: (i, 0)),
        ],
        out_shape=[
            jax.ShapeDtypeStruct((N, D), jnp.float32),
            jax.ShapeDtypeStruct((NPAD, 2 * H), jnp.float32),
        ],
    )(x, xn, w, vs, vd)


def _prew_body(ew16_ref, m_ref, o_ref):
    o_ref[...] = jnp.dot(ew16_ref[...], m_ref[...],
                         preferred_element_type=jnp.float32)


def _prew(ew16, m):
    return pl.pallas_call(
        _prew_body,
        grid=(EPAD // 16 // BLKE,),
        in_specs=[
            pl.BlockSpec((BLKE, 16), lambda i: (i, 0)),
            pl.BlockSpec((16, D), lambda i: (0, 0)),
        ],
        out_specs=pl.BlockSpec((BLKE, D), lambda i: (i, 0)),
        out_shape=jax.ShapeDtypeStruct((EPAD // 16, D), jnp.float32),
    )(ew16, m)


def _take(v, idx):
    return v.at[idx].get(mode="promise_in_bounds")


def _sc1_body(s0, d0, w0, a0, s1, d1, w1, a1,
              s2, d2, w2, a2, s3, d3, w3, a3,
              exh,
              asat_spm, as_v, ad_v, ex_v, wc_v, si_v, di_v,
              sem0, sem1):
    cid = lax.axis_index("c")
    sid = lax.axis_index("s")
    wid = sid * NC + cid
    iota = lax.iota(jnp.int32, 16)
    lo_mask = iota < 8
    perm8 = lax.bitwise_and(iota + 8, 15)
    row0 = sid * RPS
    SE = SUPK * EC

    rels = [(s0, d0, w0, a0), (s1, d1, w1, a1),
            (s2, d2, w2, a2), (s3, d3, w3, a3)]
    for r, (src2, dst2, ewce2, asat_t) in enumerate(rels):
        for pc in range(8):
            pr = pl.multiple_of(row0 + pc * (RPS // 8), RPS // 8)
            pltpu.sync_copy(asat_t.at[pl.ds(pr, RPS // 8)],
                            asat_spm.at[pl.ds(pr, RPS // 8)])
        plsc.subcore_barrier()

        def chunk_body(k, carry):
            sup = wid * S1 + k
            crow = pl.multiple_of(sup * SUPK, SUPK)
            base = pl.multiple_of(sup * SE, SE)
            pltpu.sync_copy(src2.at[pl.ds(crow, SUPK)], si_v)
            pltpu.sync_copy(dst2.at[pl.ds(crow, SUPK)], di_v)
            wrow = pl.multiple_of(sup * (SE // 16), SE // 16)
            pltpu.sync_copy(ewce2.at[pl.ds(wrow, SE // 16)], wc_v)
            for j2 in range(SUPK // 2):
                cps = [
                    pltpu.async_copy(asat_spm.at[si_v.at[2 * j2]],
                                     as_v.at[pl.ds(0, EC)], sem0),
                    pltpu.async_copy(asat_spm.at[di_v.at[2 * j2]],
                                     ad_v.at[pl.ds(0, EC)], sem1),
                    pltpu.async_copy(asat_spm.at[si_v.at[2 * j2 + 1]],
                                     as_v.at[pl.ds(EC, EC)], sem0),
                    pltpu.async_copy(asat_spm.at[di_v.at[2 * j2 + 1]],
                                     ad_v.at[pl.ds(EC, EC)], sem1),
                ]
                for cp in cps:
                    cp.wait()

                def alpha_body(g, carry2):
                    gg = pl.multiple_of(g * 16, 16)
                    for m in range(8):
                        e0 = gg + 2 * m
                        a_0 = jnp.where(lo_mask, as_v[e0], ad_v[e0])
                        a_1 = jnp.where(lo_mask, as_v[e0 + 1], ad_v[e0 + 1])
                        v = jnp.where(lo_mask, a_0, _take(a_1, perm8))
                        al = v + wc_v[j2 * 16 + g, pl.ds(m * 16, 16)]
                        al = jnp.maximum(al, 0.2 * al)
                        exv = jnp.exp(al)
                        fo = (2 * m) % 8
                        prow = j2 * 32 + g * 2 + m // 4
                        ex_v[prow, pl.ds(fo * 16, 16)] = exv
                        ex_v[prow, pl.ds((fo + 1) * 16, 16)] = _take(exv, perm8)
                    return carry2

                lax.fori_loop(0, 2 * EC // 16, alpha_body, 0)
            base8 = pl.multiple_of(sup * (SE // 8), SE // 8)
            pltpu.sync_copy(ex_v, exh.at[r, pl.ds(base8, SE // 8)])
            return carry

        lax.fori_loop(0, S1, chunk_body, 0)
        plsc.subcore_barrier()


def _sc1_call(edges1):
    mesh = plsc.VectorSubcoreMesh(core_axis_name="c", subcore_axis_name="s")
    f = pl.kernel(
        _sc1_body,
        out_type=[
            jax.ShapeDtypeStruct((4, EPAD // 8, D), jnp.float32),
        ],
        mesh=mesh,
        scratch_types=[
            pltpu.VMEM_SHARED((NPAD, 2 * H), jnp.float32),
            pltpu.VMEM((2 * EC, 2 * H), jnp.float32),
            pltpu.VMEM((2 * EC, 2 * H), jnp.float32),
            pltpu.VMEM((SUPK * EC // 8, D), jnp.float32),
            pltpu.VMEM((SUPK * EC // 16, D), jnp.float32),
            pltpu.VMEM((SUPK, EC), jnp.int32),
            pltpu.VMEM((SUPK, EC), jnp.int32),
            pltpu.SemaphoreType.DMA,
            pltpu.SemaphoreType.DMA,
        ],
    )
    flat = []
    for (src2, dst2, ewce2, asat) in edges1:
        flat += [src2, dst2, ewce2, asat]
    (exh,) = f(*flat)
    return exh


def _sc3_body(d0, d1, d2, d3, exh, zh16, denp,
              den_spm, ex_v, pk_v, di_v, sem0, sem1):
    cid = lax.axis_index("c")
    sid = lax.axis_index("s")
    wid = sid * NC + cid
    row0 = sid * RPS
    SE = SUPK * EC

    for r, dst2 in enumerate((d0, d1, d2, d3)):
        for k in range(RPS // DR):
            pltpu.sync_copy(zh16, den_spm.at[pl.ds(row0 + k * DR, DR)])
        plsc.subcore_barrier()

        def chunk_body(k, carry):
            sup = wid * S1 + k
            crow = pl.multiple_of(sup * SUPK, SUPK)
            base8 = pl.multiple_of(sup * (SE // 8), SE // 8)
            pltpu.sync_copy(dst2.at[pl.ds(crow, SUPK)], di_v)
            pltpu.sync_copy(exh.at[r, pl.ds(base8, SE // 8)], pk_v)

            for j2 in range(SUPK // 2):

                def rp_body(rr, carry2):
                    for sub in range(8):
                        ex_v[rr * 8 + sub] = pk_v[j2 * 32 + rr,
                                                  pl.ds(sub * 16, 16)]
                    return carry2

                lax.fori_loop(0, 2 * EC // 8, rp_body, 0)
                cps = [
                    pltpu.async_copy(ex_v.at[pl.ds(0, EC)],
                                     den_spm.at[di_v.at[2 * j2]],
                                     sem0, add=True),
                    pltpu.async_copy(ex_v.at[pl.ds(EC, EC)],
                                     den_spm.at[di_v.at[2 * j2 + 1]],
                                     sem1, add=True),
                ]
                for cp in cps:
                    cp.wait()
            return carry

        lax.fori_loop(0, S1, chunk_body, 0)
        plsc.subcore_barrier()

        for k in range(RPS // DR):
            rr = row0 + k * DR
            pltpu.sync_copy(den_spm.at[pl.ds(rr, DR)],
                            denp.at[r, cid, pl.ds(rr, DR)])


def _sc3_call(dsts2, exh):
    mesh = plsc.VectorSubcoreMesh(core_axis_name="c", subcore_axis_name="s")
    f = pl.kernel(
        _sc3_body,
        out_type=[
            jax.ShapeDtypeStruct((4, NC, NPAD, 2 * H), jnp.float32),
        ],
        mesh=mesh,
        scratch_types=[
            pltpu.VMEM_SHARED((NPAD, 2 * H), jnp.float32),
            pltpu.VMEM((2 * EC, 2 * H), jnp.float32),
            pltpu.VMEM((SUPK * EC // 8, D), jnp.float32),
            pltpu.VMEM((SUPK, EC), jnp.int32),
            pltpu.SemaphoreType.DMA,
            pltpu.SemaphoreType.DMA,
        ],
    )
    zh16 = jnp.zeros((DR, 2 * H), jnp.float32)
    (denp,) = f(*dsts2, exh, zh16)
    return denp


def _sc2_body(s0, d0, h0, s1, d1, h1, s2, d2, h2, s3, d3, h3,
              exh, zh, outp,
              out_spm, hs_a, hs_b, ex_a, ex_b, si_a, si_b, di_a, di_b,
              sem_a, sem_b, sem_c):
    cid = lax.axis_index("c")
    sid = lax.axis_index("s")
    wid = sid * NC + cid
    fullh = [jnp.full((16,), h, jnp.int32) for h in range(H)]
    row0 = sid * RPS
    c0 = wid * K2N

    def scale(hs_v, ex_v):
        def scale_body(rr, carry2):
            for sub in range(8):
                e = rr * 8 + sub
                field = ex_v[rr, pl.ds(sub * 16, 16)]
                for h in range(H):
                    sp = _take(field, fullh[h])
                    hs_v[e, pl.ds(h * 16, 16)] = hs_v[e, pl.ds(h * 16, 16)] * sp
            return carry2

        lax.fori_loop(0, EC // 8, scale_body, 0)

    rels = [(s0, d0, h0), (s1, d1, h1), (s2, d2, h2), (s3, d3, h3)]
    for r, (src2, dst2, hs_t) in enumerate(rels):
        for k in range(RPS // DR):
            pltpu.sync_copy(zh, out_spm.at[pl.ds(row0 + k * DR, DR)])
        plsc.subcore_barrier()

        pltpu.sync_copy(src2.at[c0], si_a)
        pltpu.sync_copy(dst2.at[c0], di_a)
        pltpu.sync_copy(exh.at[r, pl.ds(c0 * (EC // 8), EC // 8)], ex_a)
        pltpu.async_copy(hs_t.at[si_a], hs_a, sem_a)

        def chunk_body(k2, carry):
            cA = c0 + 2 * k2
            rowB = pl.multiple_of((cA + 1) * (EC // 8), EC // 8)
            pltpu.sync_copy(src2.at[cA + 1], si_b)
            pltpu.sync_copy(dst2.at[cA + 1], di_b)
            pltpu.sync_copy(exh.at[r, pl.ds(rowB, EC // 8)], ex_b)
            cpb = pltpu.async_copy(hs_t.at[si_b], hs_b, sem_b)
            pltpu.make_async_copy(hs_t.at[pl.ds(0, EC)], hs_a, sem_a).wait()
            scale(hs_a, ex_a)
            pltpu.async_copy(hs_a, out_spm.at[di_a], sem_c, add=True)

            @pl.when(k2 + 1 < K2N // 2)
            def _prefetch():
                rowA = pl.multiple_of((cA + 2) * (EC // 8), EC // 8)
                pltpu.make_async_copy(hs_t.at[pl.ds(0, EC)], hs_a,
                                      sem_c).wait()
                pltpu.sync_copy(src2.at[cA + 2], si_a)
                pltpu.sync_copy(dst2.at[cA + 2], di_a)
                pltpu.sync_copy(exh.at[r, pl.ds(rowA, EC // 8)], ex_a)
                pltpu.async_copy(hs_t.at[si_a], hs_a, sem_a)

            @pl.when(k2 + 1 >= K2N // 2)
            def _last_drain():
                pltpu.make_async_copy(hs_t.at[pl.ds(0, EC)], hs_a,
                                      sem_c).wait()

            cpb.wait()
            scale(hs_b, ex_b)
            pltpu.sync_copy(hs_b, out_spm.at[di_b], add=True)
            return carry

        lax.fori_loop(0, K2N // 2, chunk_body, 0)
        plsc.subcore_barrier()

        for k in range(RPS // DR):
            rr = row0 + k * DR
            pltpu.sync_copy(out_spm.at[pl.ds(rr, DR)],
                            outp.at[r, cid, pl.ds(rr, DR)])


def _sc2_call(edges2, exh):
    mesh = plsc.VectorSubcoreMesh(core_axis_name="c", subcore_axis_name="s")
    f = pl.kernel(
        _sc2_body,
        out_type=[
            jax.ShapeDtypeStruct((4, NC, NPAD, D), jnp.float32),
        ],
        mesh=mesh,
        scratch_types=[
            pltpu.VMEM_SHARED((NPAD, D), jnp.float32),
            pltpu.VMEM((EC, D), jnp.float32),
            pltpu.VMEM((EC, D), jnp.float32),
            pltpu.VMEM((EC // 8, D), jnp.float32),
            pltpu.VMEM((EC // 8, D), jnp.float32),
            pltpu.VMEM((EC,), jnp.int32),
            pltpu.VMEM((EC,), jnp.int32),
            pltpu.VMEM((EC,), jnp.int32),
            pltpu.VMEM((EC,), jnp.int32),
            pltpu.SemaphoreType.DMA,
            pltpu.SemaphoreType.DMA,
            pltpu.SemaphoreType.DMA,
        ],
    )
    flat = []
    for (src2, dst2, hs) in edges2:
        flat += [src2, dst2, hs]
    zh = jnp.zeros((DR, D), jnp.float32)
    (outp,) = f(*flat, exh, zh)
    return outp


def _post_body(outp_ref, denp_ref, xn_ref, u_ref, w1_ref, w2_ref, lb_ref,
               b4_ref, r16_ref, o_ref):
    xn = xn_ref[...]
    u1 = u_ref[0:1, :]
    u2 = u_ref[1:2, :]
    zx = jnp.sum(xn * u2, axis=1, keepdims=True)
    r16 = r16_ref[...]
    acc = None
    ssum = None
    for r in range(4):
        o = outp_ref[r, 0] + outp_ref[r, 1]
        dn = denp_ref[r, 0] + denp_ref[r, 1]
        denrep = jnp.dot(dn, r16, preferred_element_type=jnp.float32)
        v = o / (denrep + 1e-16) + b4_ref[r:r + 1, :]
        v = jnp.maximum(v, 0.0)
        z = jnp.sum(v * u1, axis=1, keepdims=True) + zx
        s = jnp.exp(jnp.maximum(z, 0.01 * z))
        acc = s * v if acc is None else acc + s * v
        ssum = s if ssum is None else ssum + s
    comb = acc / ssum
    hh = (jnp.dot(xn, w1_ref[...], preferred_element_type=jnp.float32)
          + jnp.dot(comb, w2_ref[...], preferred_element_type=jnp.float32)
          + lb_ref[...])
    hh = jnp.maximum(hh, 0.0)
    nrm = jnp.sqrt(jnp.sum(hh * hh, axis=1, keepdims=True))
    o_ref[...] = hh / jnp.maximum(nrm, 1e-12)


def _post(outp, denp, xn, u2d, w1, w2, lb, b4, r16):
    return pl.pallas_call(
        _post_body,
        grid=(N // BLK,),
        in_specs=[
            pl.BlockSpec((4, NC, BLK, D), lambda i: (0, 0, i, 0)),
            pl.BlockSpec((4, NC, BLK, 2 * H), lambda i: (0, 0, i, 0)),
            pl.BlockSpec((BLK, D), lambda i: (i, 0)),
            pl.BlockSpec((2, D), lambda i: (0, 0)),
            pl.BlockSpec((D, D), lambda i: (0, 0)),
            pl.BlockSpec((D, D), lambda i: (0, 0)),
            pl.BlockSpec((1, D), lambda i: (0, 0)),
            pl.BlockSpec((4, D), lambda i: (0, 0)),
            pl.BlockSpec((2 * H, D), lambda i: (0, 0)),
        ],
        out_specs=pl.BlockSpec((BLK, D), lambda i: (i, 0)),
        out_shape=jax.ShapeDtypeStruct((N, D), jnp.float32),
    )(outp, denp, xn, u2d, w1, w2, lb, b4, r16)


def kernel(x_a, x_p, x_tt, x_c, x_node,
           edge_index_a, edge_index_p, edge_index_t, edge_index_c,
           ew_a, ew_p, ew_t, ew_c,
           p_a, p_p, p_t, p_c, u, lin_W, lin_b):
    kron = jnp.asarray(_KRON)
    zpad = jnp.zeros((D, H), jnp.float32)

    edges1 = []
    edges2 = []
    b_rows = []
    for x_s, ei, ew, p in ((x_a, edge_index_a, ew_a, p_a),
                           (x_p, edge_index_p, ew_p, p_p),
                           (x_tt, edge_index_t, ew_t, p_t),
                           (x_c, edge_index_c, ew_c, p_c)):
        vs = p["W_src"] @ (p["a_src"].reshape(D, 1) * kron)
        vd = p["W_dst"] @ (p["a_dst"].reshape(D, 1) * kron)
        vs_pad = jnp.concatenate([vs, zpad], axis=1)
        vd_pad = jnp.concatenate([zpad, vd], axis=1)
        ce = (p["W_e"].reshape(H, C) * p["a_e"]).sum(-1)
        m16 = jnp.asarray(_PSEL) * jnp.tile(ce, 16)[None, :]
        srcp = jnp.concatenate([ei[1], jnp.zeros((EPAD - E,), jnp.int32)])
        dstp = jnp.concatenate(
            [ei[0], jnp.full((EPAD - E,), NPAD - 1, jnp.int32)])
        ewp = jnp.concatenate([ew, jnp.zeros((EPAD - E,), jnp.float32)])
        ewce2 = _prew(ewp.reshape(EPAD // 16, 16), m16)
        b_rows.append(p["b"])
        hs, asat = _pre(x_s, x_node, p["W_src"], vs_pad, vd_pad)
        src2d = srcp.reshape(CH, EC)
        dst2d = dstp.reshape(CH, EC)
        edges1.append((src2d, dst2d, ewce2, asat))
        edges2.append((src2d, dst2d, hs))

    exh = _sc1_call(edges1)
    denp = _sc3_call([e[1] for e in edges1], exh)
    outp = _sc2_call(edges2, exh)

    b4 = jnp.stack(b_rows)
    u2d = u.reshape(2, D)
    w1 = lin_W[:D]
    w2 = lin_W[D:]
    lb = lin_b.reshape(1, D)
    r16 = jnp.asarray(np.concatenate(
        [np.kron(np.eye(H, dtype=np.float32), np.ones((1, C), np.float32)),
         np.zeros((H, D), np.float32)], axis=0))
    return _post(outp, denp, x_node, u2d, w1, w2, lb, b4, r16)


# speedup vs baseline: 1.0516x; 1.0516x over previous
import jax
import jax.numpy as jnp
import numpy as np
from jax import lax
from jax.experimental import pallas as pl
from jax.experimental.pallas import tpu as pltpu
from jax.experimental.pallas import tpu_sc as plsc

N = 10000
E = 320000
D = 128
H = 8
C = D // H
NC = 2
NS = 16
NW = NC * NS
EC = 128
EPAD = 327680
CH = EPAD // EC
SUPK = 8
S1 = CH // (NW * SUPK)
K2N = CH // NW
NPAD = 10240
RPS = NPAD // NS
DR = 128
DENW = NPAD * H

BLK = 1000
BLKE = 8192

_KRON = np.kron(np.eye(H, dtype=np.float32), np.ones((C, 1), np.float32))
_PSEL = np.zeros((16, 128), np.float32)
for _f in range(8):
    for _l in range(16):
        _PSEL[2 * _f + (_l >= 8), _f * 16 + _l] = 1.0


def _pre_body(x_ref, xn_ref, w_ref, vs_ref, vd_ref, hs_ref, asat_ref):
    x = x_ref[...]
    hs_ref[...] = jnp.dot(x, w_ref[...], preferred_element_type=jnp.float32)
    asat_ref[...] = (
        jnp.dot(x, vs_ref[...], preferred_element_type=jnp.float32)
        + jnp.dot(xn_ref[...], vd_ref[...], preferred_element_type=jnp.float32))


def _pre(x, xn, w, vs, vd):
    return pl.pallas_call(
        _pre_body,
        grid=(N // 2000,),
        in_specs=[
            pl.BlockSpec((2000, D), lambda i: (i, 0)),
            pl.BlockSpec((2000, D), lambda i: (i, 0)),
            pl.BlockSpec((D, D), lambda i: (0, 0)),
            pl.BlockSpec((D, 2 * H), lambda i: (0, 0)),
            pl.BlockSpec((D, 2 * H), lambda i: (0, 0)),
        ],
        out_specs=[
            pl.BlockSpec((2000, D), lambda i: (i, 0)),
            pl.BlockSpec((2000, 2 * H), lambda i: (i, 0)),
        ],
        out_shape=[
            jax.ShapeDtypeStruct((N, D), jnp.float32),
            jax.ShapeDtypeStruct((NPAD, 2 * H), jnp.float32),
        ],
    )(x, xn, w, vs, vd)


def _prew_body(ew16_ref, m_ref, o_ref):
    o_ref[...] = jnp.dot(ew16_ref[...], m_ref[...],
                         preferred_element_type=jnp.float32)


def _prew(ew16, m):
    return pl.pallas_call(
        _prew_body,
        grid=(EPAD // 16 // BLKE,),
        in_specs=[
            pl.BlockSpec((BLKE, 16), lambda i: (i, 0)),
            pl.BlockSpec((16, D), lambda i: (0, 0)),
        ],
        out_specs=pl.BlockSpec((BLKE, D), lambda i: (i, 0)),
        out_shape=jax.ShapeDtypeStruct((EPAD // 16, D), jnp.float32),
    )(ew16, m)


def _take(v, idx):
    return v.at[idx].get(mode="promise_in_bounds")


def _sc1_body(s0, d0, w0, a0, s1, d1, w1, a1,
              s2, d2, w2, a2, s3, d3, w3, a3,
              exh,
              asat_spm, as_v, ad_v, ex_v, wc_v, si_v, di_v,
              sem0, sem1):
    cid = lax.axis_index("c")
    sid = lax.axis_index("s")
    wid = sid * NC + cid
    iota = lax.iota(jnp.int32, 16)
    lo_mask = iota < 8
    perm8 = lax.bitwise_and(iota + 8, 15)
    row0 = sid * RPS
    SE = SUPK * EC

    rels = [(s0, d0, w0, a0), (s1, d1, w1, a1),
            (s2, d2, w2, a2), (s3, d3, w3, a3)]
    for r, (src2, dst2, ewce2, asat_t) in enumerate(rels):
        for pc in range(8):
            pr = pl.multiple_of(row0 + pc * (RPS // 8), RPS // 8)
            pltpu.sync_copy(asat_t.at[pl.ds(pr, RPS // 8)],
                            asat_spm.at[pl.ds(pr, RPS // 8)])
        plsc.subcore_barrier()

        def chunk_body(k, carry):
            sup = wid * S1 + k
            crow = pl.multiple_of(sup * SUPK, SUPK)
            base = pl.multiple_of(sup * SE, SE)
            pltpu.sync_copy(src2.at[pl.ds(crow, SUPK)], si_v)
            pltpu.sync_copy(dst2.at[pl.ds(crow, SUPK)], di_v)
            wrow = pl.multiple_of(sup * (SE // 16), SE // 16)
            pltpu.sync_copy(ewce2.at[pl.ds(wrow, SE // 16)], wc_v)
            for j2 in range(SUPK // 2):
                cps = [
                    pltpu.async_copy(asat_spm.at[si_v.at[2 * j2]],
                                     as_v.at[pl.ds(0, EC)], sem0),
                    pltpu.async_copy(asat_spm.at[di_v.at[2 * j2]],
                                     ad_v.at[pl.ds(0, EC)], sem1),
                    pltpu.async_copy(asat_spm.at[si_v.at[2 * j2 + 1]],
                                     as_v.at[pl.ds(EC, EC)], sem0),
                    pltpu.async_copy(asat_spm.at[di_v.at[2 * j2 + 1]],
                                     ad_v.at[pl.ds(EC, EC)], sem1),
                ]
                for cp in cps:
                    cp.wait()

                def alpha_body(g, carry2):
                    gg = pl.multiple_of(g * 16, 16)
                    for m in range(8):
                        e0 = gg + 2 * m
                        a_0 = jnp.where(lo_mask, as_v[e0], ad_v[e0])
                        a_1 = jnp.where(lo_mask, as_v[e0 + 1], ad_v[e0 + 1])
                        v = jnp.where(lo_mask, a_0, _take(a_1, perm8))
                        al = v + wc_v[j2 * 16 + g, pl.ds(m * 16, 16)]
                        al = jnp.maximum(al, 0.2 * al)
                        exv = jnp.exp(al)
                        fo = (2 * m) % 8
                        prow = j2 * 32 + g * 2 + m // 4
                        ex_v[prow, pl.ds(fo * 16, 16)] = exv
                        ex_v[prow, pl.ds((fo + 1) * 16, 16)] = _take(exv, perm8)
                    return carry2

                lax.fori_loop(0, 2 * EC // 16, alpha_body, 0)
            base8 = pl.multiple_of(sup * (SE // 8), SE // 8)
            pltpu.sync_copy(ex_v, exh.at[r, pl.ds(base8, SE // 8)])
            return carry

        lax.fori_loop(0, S1, chunk_body, 0)
        plsc.subcore_barrier()


def _sc1_call(edges1):
    mesh = plsc.VectorSubcoreMesh(core_axis_name="c", subcore_axis_name="s")
    f = pl.kernel(
        _sc1_body,
        out_type=[
            jax.ShapeDtypeStruct((4, EPAD // 8, D), jnp.float32),
        ],
        mesh=mesh,
        scratch_types=[
            pltpu.VMEM_SHARED((NPAD, 2 * H), jnp.float32),
            pltpu.VMEM((2 * EC, 2 * H), jnp.float32),
            pltpu.VMEM((2 * EC, 2 * H), jnp.float32),
            pltpu.VMEM((SUPK * EC // 8, D), jnp.float32),
            pltpu.VMEM((SUPK * EC // 16, D), jnp.float32),
            pltpu.VMEM((SUPK, EC), jnp.int32),
            pltpu.VMEM((SUPK, EC), jnp.int32),
            pltpu.SemaphoreType.DMA,
            pltpu.SemaphoreType.DMA,
        ],
    )
    flat = []
    for (src2, dst2, ewce2, asat) in edges1:
        flat += [src2, dst2, ewce2, asat]
    (exh,) = f(*flat)
    return exh


def _sc3_body(d0, d1, d2, d3, exh, zh16, denp,
              den_spm, ex_v, pk_v, di_v, sem0, sem1):
    cid = lax.axis_index("c")
    sid = lax.axis_index("s")
    wid = sid * NC + cid
    row0 = sid * RPS
    SE = SUPK * EC

    for r, dst2 in enumerate((d0, d1, d2, d3)):
        for k in range(RPS // DR):
            pltpu.sync_copy(zh16, den_spm.at[pl.ds(row0 + k * DR, DR)])
        plsc.subcore_barrier()

        def chunk_body(k, carry):
            sup = wid * S1 + k
            crow = pl.multiple_of(sup * SUPK, SUPK)
            base8 = pl.multiple_of(sup * (SE // 8), SE // 8)
            pltpu.sync_copy(dst2.at[pl.ds(crow, SUPK)], di_v)
            pltpu.sync_copy(exh.at[r, pl.ds(base8, SE // 8)], pk_v)

            for j2 in range(SUPK // 2):

                def rp_body(rr, carry2):
                    for sub in range(8):
                        ex_v[rr * 8 + sub] = pk_v[j2 * 32 + rr,
                                                  pl.ds(sub * 16, 16)]
                    return carry2

                lax.fori_loop(0, 2 * EC // 8, rp_body, 0)
                cps = [
                    pltpu.async_copy(ex_v.at[pl.ds(0, EC)],
                                     den_spm.at[di_v.at[2 * j2]],
                                     sem0, add=True),
                    pltpu.async_copy(ex_v.at[pl.ds(EC, EC)],
                                     den_spm.at[di_v.at[2 * j2 + 1]],
                                     sem1, add=True),
                ]
                for cp in cps:
                    cp.wait()
            return carry

        lax.fori_loop(0, S1, chunk_body, 0)
        plsc.subcore_barrier()

        for k in range(RPS // DR):
            rr = row0 + k * DR
            pltpu.sync_copy(den_spm.at[pl.ds(rr, DR)],
                            denp.at[r, cid, pl.ds(rr, DR)])


def _sc3_call(dsts2, exh):
    mesh = plsc.VectorSubcoreMesh(core_axis_name="c", subcore_axis_name="s")
    f = pl.kernel(
        _sc3_body,
        out_type=[
            jax.ShapeDtypeStruct((4, NC, NPAD, 2 * H), jnp.float32),
        ],
        mesh=mesh,
        scratch_types=[
            pltpu.VMEM_SHARED((NPAD, 2 * H), jnp.float32),
            pltpu.VMEM((2 * EC, 2 * H), jnp.float32),
            pltpu.VMEM((SUPK * EC // 8, D), jnp.float32),
            pltpu.VMEM((SUPK, EC), jnp.int32),
            pltpu.SemaphoreType.DMA,
            pltpu.SemaphoreType.DMA,
        ],
    )
    zh16 = jnp.zeros((DR, 2 * H), jnp.float32)
    (denp,) = f(*dsts2, exh, zh16)
    return denp


def _sc2_body(s0, d0, h0, s1, d1, h1, s2, d2, h2, s3, d3, h3,
              exh, zh, outp,
              out_spm, hs_a, hs_b, ex_a, ex_b, si_a, si_b, di_a, di_b,
              sem_a, sem_b):
    cid = lax.axis_index("c")
    sid = lax.axis_index("s")
    wid = sid * NC + cid
    fullh = [jnp.full((16,), h, jnp.int32) for h in range(H)]
    row0 = sid * RPS
    c0 = wid * K2N

    def scale(hs_v, ex_v):
        def scale_body(rr, carry2):
            for sub in range(8):
                e = rr * 8 + sub
                field = ex_v[rr, pl.ds(sub * 16, 16)]
                for h in range(H):
                    sp = _take(field, fullh[h])
                    hs_v[e, pl.ds(h * 16, 16)] = hs_v[e, pl.ds(h * 16, 16)] * sp
            return carry2

        lax.fori_loop(0, EC // 8, scale_body, 0)

    rels = [(s0, d0, h0), (s1, d1, h1), (s2, d2, h2), (s3, d3, h3)]
    for r, (src1, dst1, hs_t) in enumerate(rels):
        for k in range(RPS // DR):
            pltpu.sync_copy(zh, out_spm.at[pl.ds(row0 + k * DR, DR)])
        plsc.subcore_barrier()

        baseA0 = pl.multiple_of(c0 * EC, EC)
        pltpu.sync_copy(src1.at[pl.ds(baseA0, EC)], si_a)
        pltpu.sync_copy(dst1.at[pl.ds(baseA0, EC)], di_a)
        pltpu.sync_copy(exh.at[r, pl.ds(c0 * (EC // 8), EC // 8)], ex_a)
        pltpu.async_copy(hs_t.at[si_a], hs_a, sem_a)

        def chunk_body(k2, carry):
            cA = c0 + 2 * k2
            rowB = pl.multiple_of((cA + 1) * (EC // 8), EC // 8)
            baseB = pl.multiple_of((cA + 1) * EC, EC)
            pltpu.sync_copy(src1.at[pl.ds(baseB, EC)], si_b)
            pltpu.sync_copy(dst1.at[pl.ds(baseB, EC)], di_b)
            pltpu.sync_copy(exh.at[r, pl.ds(rowB, EC // 8)], ex_b)
            cpb = pltpu.async_copy(hs_t.at[si_b], hs_b, sem_b)
            pltpu.make_async_copy(hs_t.at[pl.ds(0, EC)], hs_a, sem_a).wait()
            scale(hs_a, ex_a)
            pltpu.sync_copy(hs_a, out_spm.at[di_a], add=True)

            @pl.when(k2 + 1 < K2N // 2)
            def _prefetch():
                rowA = pl.multiple_of((cA + 2) * (EC // 8), EC // 8)
                baseA = pl.multiple_of((cA + 2) * EC, EC)
                pltpu.sync_copy(src1.at[pl.ds(baseA, EC)], si_a)
                pltpu.sync_copy(dst1.at[pl.ds(baseA, EC)], di_a)
                pltpu.sync_copy(exh.at[r, pl.ds(rowA, EC // 8)], ex_a)
                pltpu.async_copy(hs_t.at[si_a], hs_a, sem_a)

            cpb.wait()
            scale(hs_b, ex_b)
            pltpu.sync_copy(hs_b, out_spm.at[di_b], add=True)
            return carry

        lax.fori_loop(0, K2N // 2, chunk_body, 0)
        plsc.subcore_barrier()

        for k in range(RPS // DR):
            rr = row0 + k * DR
            pltpu.sync_copy(out_spm.at[pl.ds(rr, DR)],
                            outp.at[r, cid, pl.ds(rr, DR)])


def _sc2_call(edges2, exh):
    mesh = plsc.VectorSubcoreMesh(core_axis_name="c", subcore_axis_name="s")
    f = pl.kernel(
        _sc2_body,
        out_type=[
            jax.ShapeDtypeStruct((4, NC, NPAD, D), jnp.float32),
        ],
        mesh=mesh,
        scratch_types=[
            pltpu.VMEM_SHARED((NPAD, D), jnp.float32),
            pltpu.VMEM((EC, D), jnp.float32),
            pltpu.VMEM((EC, D), jnp.float32),
            pltpu.VMEM((EC // 8, D), jnp.float32),
            pltpu.VMEM((EC // 8, D), jnp.float32),
            pltpu.VMEM((EC,), jnp.int32),
            pltpu.VMEM((EC,), jnp.int32),
            pltpu.VMEM((EC,), jnp.int32),
            pltpu.VMEM((EC,), jnp.int32),
            pltpu.SemaphoreType.DMA,
            pltpu.SemaphoreType.DMA,
        ],
    )
    flat = []
    for (src2, dst2, hs) in edges2:
        flat += [src2, dst2, hs]
    zh = jnp.zeros((DR, D), jnp.float32)
    (outp,) = f(*flat, exh, zh)
    return outp


def _post_body(outp_ref, denp_ref, xn_ref, u_ref, w1_ref, w2_ref, lb_ref,
               b4_ref, r16_ref, o_ref):
    xn = xn_ref[...]
    u1 = u_ref[0:1, :]
    u2 = u_ref[1:2, :]
    zx = jnp.sum(xn * u2, axis=1, keepdims=True)
    r16 = r16_ref[...]
    acc = None
    ssum = None
    for r in range(4):
        o = outp_ref[r, 0] + outp_ref[r, 1]
        dn = denp_ref[r, 0] + denp_ref[r, 1]
        denrep = jnp.dot(dn, r16, preferred_element_type=jnp.float32)
        v = o / (denrep + 1e-16) + b4_ref[r:r + 1, :]
        v = jnp.maximum(v, 0.0)
        z = jnp.sum(v * u1, axis=1, keepdims=True) + zx
        s = jnp.exp(jnp.maximum(z, 0.01 * z))
        acc = s * v if acc is None else acc + s * v
        ssum = s if ssum is None else ssum + s
    comb = acc / ssum
    hh = (jnp.dot(xn, w1_ref[...], preferred_element_type=jnp.float32)
          + jnp.dot(comb, w2_ref[...], preferred_element_type=jnp.float32)
          + lb_ref[...])
    hh = jnp.maximum(hh, 0.0)
    nrm = jnp.sqrt(jnp.sum(hh * hh, axis=1, keepdims=True))
    o_ref[...] = hh / jnp.maximum(nrm, 1e-12)


def _post(outp, denp, xn, u2d, w1, w2, lb, b4, r16):
    return pl.pallas_call(
        _post_body,
        grid=(N // BLK,),
        in_specs=[
            pl.BlockSpec((4, NC, BLK, D), lambda i: (0, 0, i, 0)),
            pl.BlockSpec((4, NC, BLK, 2 * H), lambda i: (0, 0, i, 0)),
            pl.BlockSpec((BLK, D), lambda i: (i, 0)),
            pl.BlockSpec((2, D), lambda i: (0, 0)),
            pl.BlockSpec((D, D), lambda i: (0, 0)),
            pl.BlockSpec((D, D), lambda i: (0, 0)),
            pl.BlockSpec((1, D), lambda i: (0, 0)),
            pl.BlockSpec((4, D), lambda i: (0, 0)),
            pl.BlockSpec((2 * H, D), lambda i: (0, 0)),
        ],
        out_specs=pl.BlockSpec((BLK, D), lambda i: (i, 0)),
        out_shape=jax.ShapeDtypeStruct((N, D), jnp.float32),
    )(outp, denp, xn, u2d, w1, w2, lb, b4, r16)


def kernel(x_a, x_p, x_tt, x_c, x_node,
           edge_index_a, edge_index_p, edge_index_t, edge_index_c,
           ew_a, ew_p, ew_t, ew_c,
           p_a, p_p, p_t, p_c, u, lin_W, lin_b):
    kron = jnp.asarray(_KRON)
    zpad = jnp.zeros((D, H), jnp.float32)

    edges1 = []
    edges2 = []
    b_rows = []
    for x_s, ei, ew, p in ((x_a, edge_index_a, ew_a, p_a),
                           (x_p, edge_index_p, ew_p, p_p),
                           (x_tt, edge_index_t, ew_t, p_t),
                           (x_c, edge_index_c, ew_c, p_c)):
        vs = p["W_src"] @ (p["a_src"].reshape(D, 1) * kron)
        vd = p["W_dst"] @ (p["a_dst"].reshape(D, 1) * kron)
        vs_pad = jnp.concatenate([vs, zpad], axis=1)
        vd_pad = jnp.concatenate([zpad, vd], axis=1)
        ce = (p["W_e"].reshape(H, C) * p["a_e"]).sum(-1)
        m16 = jnp.asarray(_PSEL) * jnp.tile(ce, 16)[None, :]
        srcp = jnp.concatenate([ei[1], jnp.zeros((EPAD - E,), jnp.int32)])
        dstp = jnp.concatenate(
            [ei[0], jnp.full((EPAD - E,), NPAD - 1, jnp.int32)])
        ewp = jnp.concatenate([ew, jnp.zeros((EPAD - E,), jnp.float32)])
        ewce2 = _prew(ewp.reshape(EPAD // 16, 16), m16)
        b_rows.append(p["b"])
        hs, asat = _pre(x_s, x_node, p["W_src"], vs_pad, vd_pad)
        src2d = srcp.reshape(CH, EC)
        dst2d = dstp.reshape(CH, EC)
        edges1.append((src2d, dst2d, ewce2, asat))
        edges2.append((srcp, dstp, hs))

    exh = _sc1_call(edges1)
    denp = _sc3_call([e[1] for e in edges1], exh)
    outp = _sc2_call(edges2, exh)

    b4 = jnp.stack(b_rows)
    u2d = u.reshape(2, D)
    w1 = lin_W[:D]
    w2 = lin_W[D:]
    lb = lin_b.reshape(1, D)
    r16 = jnp.asarray(np.concatenate(
        [np.kron(np.eye(H, dtype=np.float32), np.ones((1, C), np.float32)),
         np.zeros((H, D), np.float32)], axis=0))
    return _post(outp, denp, x_node, u2d, w1, w2, lb, b4, r16)
